# baseline (speedup 1.0000x reference)
import jax
import jax.numpy as jnp
from jax import lax
from jax.experimental import pallas as pl
from jax.experimental.pallas import tpu as pltpu
from jax.experimental.pallas import tpu_sc as plsc

N = 10000
E = 160000
D = 256
H = 128
NC = 2
NS = 16
L = 16
K = 128
CH = 80
CHA = CH + 2
EPT = CH * K
RPT = N // NS
WR = 125
WCH = RPT // WR


def _mm_body(x_ref, w_ref, o_ref):
    o_ref[...] = jnp.dot(x_ref[...], w_ref[...],
                         preferred_element_type=jnp.float32)


def _matmul(x, weight):
    bn = 1000
    return pl.pallas_call(
        _mm_body,
        grid=(N // bn,),
        in_specs=[
            pl.BlockSpec((bn, D), lambda i: (i, 0)),
            pl.BlockSpec((D, D), lambda i: (0, 0)),
        ],
        out_specs=pl.BlockSpec((bn, D), lambda i: (i, 0)),
        out_shape=jax.ShapeDtypeStruct((N, D), jnp.float32),
    )(x, weight)


def _sc_body(xw2_hbm, src_hbm, dst_hbm, ew_hbm, widx_hbm, out_hbm,
             src_v, dst_v, widx_v, buf, ewv, acc):
    c = lax.axis_index("c")
    s = lax.axis_index("s")
    base = s * RPT
    wbuf = buf.at[pl.ds(0, WR)]

    pltpu.sync_copy(src_hbm.at[c, s], src_v)
    pltpu.sync_copy(dst_hbm.at[s], dst_v)
    pltpu.sync_copy(widx_hbm.at[c, s], widx_v)

    def ew_start(j, p):
        pltpu.sync_copy(ew_hbm.at[s, j], ewv)

    zeros = jnp.zeros((L,), jnp.float32)

    def zrow(r, _):
        for v in range(H // L):
            buf[r, pl.ds(v * L, L)] = zeros
        return 0

    lax.fori_loop(0, WR, zrow, 0)

    def zcopy(k, _):
        pltpu.sync_copy(wbuf, acc.at[pl.ds(base + k * WR, WR)])
        return 0

    lax.fori_loop(0, WCH, zcopy, 0)
    plsc.subcore_barrier()

    def scale(p):
        def edge_body(g, _):
            for u in range(2):
                e = g * 2 + u
                w = ewv[pl.ds(e * L, L)]
                for v in range(H // L):
                    sl = pl.ds(v * L, L)
                    buf[e, sl] = buf[e, sl] * w
            return 0

        lax.fori_loop(0, K // 2, edge_body, 0)

    def chunk_body(j, _):
        pltpu.sync_copy(xw2_hbm.at[src_v.at[j]], buf)
        ew_start(j, 0)
        scale(0)
        pltpu.sync_copy(buf, acc.at[dst_v.at[j]], add=True)
        return 0

    lax.fori_loop(0, CH, chunk_body, 0)
    plsc.subcore_barrier()

    def wo_body(k, _):
        pltpu.sync_copy(acc.at[pl.ds(base + k * WR, WR)], wbuf)

        def relu_row(r, _):
            for v in range(H // L):
                sl = pl.ds(v * L, L)
                buf[r, sl] = jnp.maximum(buf[r, sl], 0.0)
            return 0

        lax.fori_loop(0, WR, relu_row, 0)
        pltpu.sync_copy(wbuf, out_hbm.at[widx_v.at[k]])
        return 0

    lax.fori_loop(0, WCH, wo_body, 0)


def _sc_scatter(xw2, src2, dst3, ew3, widx):
    mesh = plsc.VectorSubcoreMesh(core_axis_name="c", subcore_axis_name="s",
                                  num_cores=NC, num_subcores=NS)
    return pl.kernel(
        _sc_body,
        out_type=jax.ShapeDtypeStruct((2 * N, H), jnp.float32),
        mesh=mesh,
        scratch_types=[
            pltpu.VMEM((CH, K), jnp.int32),
            pltpu.VMEM((CH, K), jnp.int32),
            pltpu.VMEM((WCH, WR), jnp.int32),
            pltpu.VMEM((K, H), jnp.float32),
            pltpu.VMEM((K * L,), jnp.float32),
            pltpu.VMEM_SHARED((N, H), jnp.float32),
        ],
    )(xw2, src2, dst3, ew3, widx)


def kernel(x, edge_index, edge_weight, weight):
    xw = _matmul(x, weight)
    xw2 = xw.reshape(2 * N, H)

    ept = E // NS
    srcp = jnp.pad(edge_index[0].reshape(NS, ept), ((0, 0), (0, EPT - ept)))
    dstp = jnp.pad(edge_index[1].reshape(NS, ept), ((0, 0), (0, EPT - ept)))
    ewp = jnp.pad(edge_weight.reshape(NS, ept), ((0, 0), (0, EPT - ept)))

    core = jnp.arange(NC, dtype=jnp.int32)
    src2 = (2 * srcp[None] + core[:, None, None]).reshape(NC, NS, CH, K)
    dst3 = dstp.reshape(NS, CH, K)
    ew3 = jnp.broadcast_to(ewp[:, :, None],
                           (NS, EPT, L)).reshape(NS, CH, K * L)

    rows = jnp.arange(N, dtype=jnp.int32).reshape(NS, WCH, WR)
    widx = 2 * rows[None] + core[:, None, None, None]

    out_flat = _sc_scatter(xw2, src2, dst3, ew3, widx)
    return out_flat.reshape(N, D)


# speedup vs baseline: 1.0248x; 1.0248x over previous
import jax
import jax.numpy as jnp
from jax import lax
from jax.experimental import pallas as pl
from jax.experimental.pallas import tpu as pltpu
from jax.experimental.pallas import tpu_sc as plsc

N = 10000
E = 160000
D = 256
H = 128
NC = 2
NS = 16
L = 16
K = 128
CH = 79
EPT = CH * K
RPT = N // NS
WR = 125
WCH = RPT // WR


def _mm_body(x_ref, w_ref, o_ref):
    o_ref[...] = jnp.dot(x_ref[...], w_ref[...],
                         preferred_element_type=jnp.float32)


def _matmul(x, weight):
    bn = 1000
    return pl.pallas_call(
        _mm_body,
        grid=(N // bn,),
        in_specs=[
            pl.BlockSpec((bn, D), lambda i: (i, 0)),
            pl.BlockSpec((D, D), lambda i: (0, 0)),
        ],
        out_specs=pl.BlockSpec((bn, D), lambda i: (i, 0)),
        out_shape=jax.ShapeDtypeStruct((N, D), jnp.float32),
    )(x, weight)


def _sc_body(xw2_hbm, src_hbm, dst_hbm, ew_hbm, widx_hbm, out_hbm,
             src_v, dst_v, widx_v, buf, ewv, acc):
    c = lax.axis_index("c")
    s = lax.axis_index("s")
    base = s * RPT
    wbuf = buf.at[pl.ds(0, WR)]

    pltpu.sync_copy(src_hbm.at[c, s], src_v)
    pltpu.sync_copy(dst_hbm.at[s], dst_v)
    pltpu.sync_copy(widx_hbm.at[c, s], widx_v)

    def ew_start(j, p):
        pltpu.sync_copy(ew_hbm.at[s, j], ewv)

    zeros = jnp.zeros((L,), jnp.float32)

    def zrow(r, _):
        for v in range(H // L):
            buf[r, pl.ds(v * L, L)] = zeros
        return 0

    lax.fori_loop(0, WR, zrow, 0)

    def zcopy(k, _):
        pltpu.sync_copy(wbuf, acc.at[pl.ds(base + k * WR, WR)])
        return 0

    lax.fori_loop(0, WCH, zcopy, 0)
    plsc.subcore_barrier()

    def scale(p):
        def edge_body(e, _):
            w = ewv[pl.ds(e * L, L)]
            for v in range(H // L):
                sl = pl.ds(v * L, L)
                buf[e, sl] = buf[e, sl] * w
            return 0

        lax.fori_loop(0, K, edge_body, 0)

    def chunk_body(j, _):
        pltpu.sync_copy(xw2_hbm.at[src_v.at[j]], buf)
        ew_start(j, 0)
        scale(0)
        pltpu.sync_copy(buf, acc.at[dst_v.at[j]], add=True)
        return 0

    lax.fori_loop(0, CH, chunk_body, 0)
    plsc.subcore_barrier()

    def wo_body(k, _):
        pltpu.sync_copy(acc.at[pl.ds(base + k * WR, WR)], wbuf)

        def relu_row(r, _):
            for v in range(H // L):
                sl = pl.ds(v * L, L)
                buf[r, sl] = jnp.maximum(buf[r, sl], 0.0)
            return 0

        lax.fori_loop(0, WR, relu_row, 0)
        pltpu.sync_copy(wbuf, out_hbm.at[widx_v.at[k]])
        return 0

    lax.fori_loop(0, WCH, wo_body, 0)


def _sc_scatter(xw2, src2, dst3, ew3, widx):
    mesh = plsc.VectorSubcoreMesh(core_axis_name="c", subcore_axis_name="s",
                                  num_cores=NC, num_subcores=NS)
    return pl.kernel(
        _sc_body,
        out_type=jax.ShapeDtypeStruct((2 * N, H), jnp.float32),
        mesh=mesh,
        scratch_types=[
            pltpu.VMEM((CH, K), jnp.int32),
            pltpu.VMEM((CH, K), jnp.int32),
            pltpu.VMEM((WCH, WR), jnp.int32),
            pltpu.VMEM((K, H), jnp.float32),
            pltpu.VMEM((K * L,), jnp.float32),
            pltpu.VMEM_SHARED((N, H), jnp.float32),
        ],
    )(xw2, src2, dst3, ew3, widx)


def kernel(x, edge_index, edge_weight, weight):
    xw = _matmul(x, weight)
    xw2 = xw.reshape(2 * N, H)

    ept = E // NS
    srcp = jnp.pad(edge_index[0].reshape(NS, ept), ((0, 0), (0, EPT - ept)))
    dstp = jnp.pad(edge_index[1].reshape(NS, ept), ((0, 0), (0, EPT - ept)))
    ewp = jnp.pad(edge_weight.reshape(NS, ept), ((0, 0), (0, EPT - ept)))

    core = jnp.arange(NC, dtype=jnp.int32)
    src2 = (2 * srcp[None] + core[:, None, None]).reshape(NC, NS, CH, K)
    dst3 = dstp.reshape(NS, CH, K)
    ew3 = jnp.broadcast_to(ewp[:, :, None],
                           (NS, EPT, L)).reshape(NS, CH, K * L)

    rows = jnp.arange(N, dtype=jnp.int32).reshape(NS, WCH, WR)
    widx = 2 * rows[None] + core[:, None, None, None]

    out_flat = _sc_scatter(xw2, src2, dst3, ew3, widx)
    return out_flat.reshape(N, D)


# speedup vs baseline: 1.2690x; 1.2383x over previous
import jax
import jax.numpy as jnp
from jax import lax
from jax.experimental import pallas as pl
from jax.experimental.pallas import tpu as pltpu
from jax.experimental.pallas import tpu_sc as plsc

N = 10000
E = 160000
D = 256
H = 128
NC = 2
NS = 16
L = 16
K = 128
CH = 79
EPT = CH * K
RPT = N // NS
WR = 125
WCH = RPT // WR


def _mm_body(x_ref, w_ref, o_ref):
    o_ref[...] = jnp.dot(x_ref[...], w_ref[...],
                         preferred_element_type=jnp.float32)


def _matmul(x, weight):
    bn = 1000
    return pl.pallas_call(
        _mm_body,
        grid=(N // bn,),
        in_specs=[
            pl.BlockSpec((bn, D), lambda i: (i, 0)),
            pl.BlockSpec((D, D), lambda i: (0, 0)),
        ],
        out_specs=pl.BlockSpec((bn, D), lambda i: (i, 0)),
        out_shape=jax.ShapeDtypeStruct((N, D), jnp.float32),
    )(x, weight)


def _sc_body(xw2_hbm, src_hbm, dst_hbm, ew_hbm, widx_hbm, out_hbm,
             src_v, dst_v, ewv, widx_v, buf, acc):
    c = lax.axis_index("c")
    s = lax.axis_index("s")
    base = s * RPT
    wbuf = buf.at[pl.ds(0, WR)]

    pltpu.sync_copy(src_hbm.at[c, s], src_v)
    pltpu.sync_copy(dst_hbm.at[s], dst_v)
    pltpu.sync_copy(widx_hbm.at[c, s], widx_v)

    def ew_start(j, p):
        pltpu.sync_copy(ew_hbm.at[s, j], ewv)

    zeros = jnp.zeros((L,), jnp.float32)

    def zrow(r, _):
        for v in range(H // L):
            buf[r, pl.ds(v * L, L)] = zeros
        return 0

    lax.fori_loop(0, WR, zrow, 0)

    def zcopy(k, _):
        pltpu.sync_copy(wbuf, acc.at[pl.ds(base + k * WR, WR)])
        return 0

    lax.fori_loop(0, WCH, zcopy, 0)
    plsc.subcore_barrier()

    def scale(p):
        def edge_body(e, _):
            w = ewv[pl.ds(e * L, L)]
            for v in range(H // L):
                sl = pl.ds(v * L, L)
                buf[e, sl] = buf[e, sl] * w
            return 0

        lax.fori_loop(0, K, edge_body, 0)

    def chunk_body(j, _):
        pltpu.sync_copy(xw2_hbm.at[src_v.at[j]], buf)
        ew_start(j, 0)
        scale(0)
        pltpu.sync_copy(buf, acc.at[dst_v.at[j]], add=True)
        return 0

    lax.fori_loop(0, CH, chunk_body, 0)
    plsc.subcore_barrier()

    def wo_body(k, _):
        pltpu.sync_copy(acc.at[pl.ds(base + k * WR, WR)], wbuf)

        def relu_row(r, _):
            for v in range(H // L):
                sl = pl.ds(v * L, L)
                buf[r, sl] = jnp.maximum(buf[r, sl], 0.0)
            return 0

        lax.fori_loop(0, WR, relu_row, 0)
        pltpu.sync_copy(wbuf, out_hbm.at[widx_v.at[k]])
        return 0

    lax.fori_loop(0, WCH, wo_body, 0)


def _sc_scatter(xw2, src2, dst3, ew3, widx):
    mesh = plsc.VectorSubcoreMesh(core_axis_name="c", subcore_axis_name="s",
                                  num_cores=NC, num_subcores=NS)
    return pl.kernel(
        _sc_body,
        out_type=jax.ShapeDtypeStruct((2 * N, H), jnp.float32),
        mesh=mesh,
        scratch_types=[
            pltpu.VMEM((CH, K), jnp.int32),
            pltpu.VMEM((CH, K), jnp.int32),
            pltpu.VMEM((K * L,), jnp.float32),
            pltpu.VMEM((WCH, WR), jnp.int32),
            pltpu.VMEM((K, H), jnp.float32),
            pltpu.VMEM_SHARED((N, H), jnp.float32),
        ],
    )(xw2, src2, dst3, ew3, widx)


def kernel(x, edge_index, edge_weight, weight):
    xw = _matmul(x, weight)
    xw2 = xw.reshape(2 * N, H)

    pad = NS * EPT - E
    srcp = jnp.pad(edge_index[0], (0, pad))
    dstp = jnp.pad(edge_index[1], (0, pad))
    ewp = jnp.pad(edge_weight, (0, pad))

    core = jnp.arange(NC, dtype=jnp.int32)
    src2 = (2 * srcp[None, :] + core[:, None]).reshape(NC, NS, CH, K)
    dst3 = dstp.reshape(NS, CH, K)
    ew3 = jnp.broadcast_to(ewp[:, None],
                           (NS * EPT, L)).reshape(NS, CH, K * L)

    rows = jnp.arange(N, dtype=jnp.int32).reshape(NS, WCH, WR)
    widx = 2 * rows[None] + core[:, None, None, None]

    out_flat = _sc_scatter(xw2, src2, dst3, ew3, widx)
    return out_flat.reshape(N, D)


# speedup vs baseline: 1.4010x; 1.1040x over previous
import jax
import jax.numpy as jnp
from jax import lax
from jax.experimental import pallas as pl
from jax.experimental.pallas import tpu as pltpu
from jax.experimental.pallas import tpu_sc as plsc

N = 10000
E = 160000
D = 256
H = 128
NC = 2
NS = 16
L = 16
K = 128
CH = 79
EPT = CH * K
RPT = N // NS
WR = 125
WCH = RPT // WR


def _mm_body(x_ref, w_ref, o_ref):
    o_ref[...] = jnp.dot(x_ref[...], w_ref[...],
                         preferred_element_type=jnp.float32)


def _matmul(x, weight):
    bn = 1000
    return pl.pallas_call(
        _mm_body,
        grid=(N // bn,),
        in_specs=[
            pl.BlockSpec((bn, D), lambda i: (i, 0)),
            pl.BlockSpec((D, D), lambda i: (0, 0)),
        ],
        out_specs=pl.BlockSpec((bn, D), lambda i: (i, 0)),
        out_shape=jax.ShapeDtypeStruct((N, D), jnp.float32),
    )(x, weight)


def _sc_body(xw2_hbm, src_hbm, dst_hbm, ew_hbm, widx_hbm, out_hbm,
             src_v, dst_v, ewv, widx_v, buf, acc, esem):
    c = lax.axis_index("c")
    s = lax.axis_index("s")
    base = s * RPT
    wbuf = buf.at[pl.ds(0, WR)]

    pltpu.sync_copy(src_hbm.at[c, s], src_v)
    pltpu.sync_copy(dst_hbm.at[s], dst_v)
    pltpu.sync_copy(widx_hbm.at[c, s], widx_v)

    zeros = jnp.zeros((L,), jnp.float32)

    def zrow(r, _):
        for v in range(H // L):
            buf[r, pl.ds(v * L, L)] = zeros
        return 0

    lax.fori_loop(0, WR, zrow, 0)

    def zcopy(k, _):
        pltpu.sync_copy(wbuf, acc.at[pl.ds(base + k * WR, WR)])
        return 0

    lax.fori_loop(0, WCH, zcopy, 0)
    plsc.subcore_barrier()

    def scale(p):
        def edge_body(e, _):
            w = ewv[pl.ds(e * L, L)]
            for v in range(H // L):
                sl = pl.ds(v * L, L)
                buf[e, sl] = buf[e, sl] * w
            return 0

        lax.fori_loop(0, K, edge_body, 0)

    def chunk_body(j, _):
        pltpu.async_copy(ew_hbm.at[s, j], ewv, esem)
        pltpu.sync_copy(xw2_hbm.at[src_v.at[j]], buf)
        pltpu.make_async_copy(ew_hbm.at[s, 0], ewv, esem).wait()
        scale(0)
        pltpu.sync_copy(buf, acc.at[dst_v.at[j]], add=True)
        return 0

    lax.fori_loop(0, CH, chunk_body, 0)
    plsc.subcore_barrier()

    def wo_body(k, _):
        pltpu.sync_copy(acc.at[pl.ds(base + k * WR, WR)], wbuf)

        def relu_row(r, _):
            for v in range(H // L):
                sl = pl.ds(v * L, L)
                buf[r, sl] = jnp.maximum(buf[r, sl], 0.0)
            return 0

        lax.fori_loop(0, WR, relu_row, 0)
        pltpu.sync_copy(wbuf, out_hbm.at[widx_v.at[k]])
        return 0

    lax.fori_loop(0, WCH, wo_body, 0)


def _sc_scatter(xw2, src2, dst3, ew3, widx):
    mesh = plsc.VectorSubcoreMesh(core_axis_name="c", subcore_axis_name="s",
                                  num_cores=NC, num_subcores=NS)
    return pl.kernel(
        _sc_body,
        out_type=jax.ShapeDtypeStruct((2 * N, H), jnp.float32),
        mesh=mesh,
        scratch_types=[
            pltpu.VMEM((CH, K), jnp.int32),
            pltpu.VMEM((CH, K), jnp.int32),
            pltpu.VMEM((K * L,), jnp.float32),
            pltpu.VMEM((WCH, WR), jnp.int32),
            pltpu.VMEM((K, H), jnp.float32),
            pltpu.VMEM_SHARED((N, H), jnp.float32),
            pltpu.SemaphoreType.DMA,
        ],
    )(xw2, src2, dst3, ew3, widx)


def kernel(x, edge_index, edge_weight, weight):
    xw = _matmul(x, weight)
    xw2 = xw.reshape(2 * N, H)

    pad = NS * EPT - E
    srcp = jnp.pad(edge_index[0], (0, pad))
    dstp = jnp.pad(edge_index[1], (0, pad))
    ewp = jnp.pad(edge_weight, (0, pad))

    core = jnp.arange(NC, dtype=jnp.int32)
    src2 = (2 * srcp[None, :] + core[:, None]).reshape(NC, NS, CH, K)
    dst3 = dstp.reshape(NS, CH, K)
    ew3 = jnp.broadcast_to(ewp[:, None],
                           (NS * EPT, L)).reshape(NS, CH, K * L)

    rows = jnp.arange(N, dtype=jnp.int32).reshape(NS, WCH, WR)
    widx = 2 * rows[None] + core[:, None, None, None]

    out_flat = _sc_scatter(xw2, src2, dst3, ew3, widx)
    return out_flat.reshape(N, D)


# speedup vs baseline: 1.4077x; 1.0048x over previous
import jax
import jax.numpy as jnp
from jax import lax
from jax.experimental import pallas as pl
from jax.experimental.pallas import tpu as pltpu
from jax.experimental.pallas import tpu_sc as plsc

N = 10000
E = 160000
D = 256
H = 128
NC = 2
NS = 16
L = 16
K = 128
CH = 79
EPT = CH * K
RPT = N // NS
WR = 125
WCH = RPT // WR


def _mm_body(x_ref, w_ref, o_ref):
    o_ref[...] = jnp.dot(x_ref[...], w_ref[...],
                         preferred_element_type=jnp.float32)


def _matmul(x, weight):
    bn = 1000
    return pl.pallas_call(
        _mm_body,
        grid=(N // bn,),
        in_specs=[
            pl.BlockSpec((bn, D), lambda i: (i, 0)),
            pl.BlockSpec((D, D), lambda i: (0, 0)),
        ],
        out_specs=pl.BlockSpec((bn, D), lambda i: (i, 0)),
        out_shape=jax.ShapeDtypeStruct((N, D), jnp.float32),
    )(x, weight)


def _sc_body(xw2_hbm, src_hbm, dst_hbm, ew_hbm, widx_hbm, out_hbm,
             src_v, dst_v, ewv, widx_v, buf, acc, esem):
    c = lax.axis_index("c")
    s = lax.axis_index("s")
    base = s * RPT
    wbuf = buf.at[pl.ds(0, WR)]

    pltpu.async_copy(src_hbm.at[c, s], src_v, esem)
    pltpu.async_copy(dst_hbm.at[s], dst_v, esem)
    pltpu.async_copy(widx_hbm.at[c, s], widx_v, esem)
    pltpu.make_async_copy(src_hbm.at[c, s], src_v, esem).wait()
    pltpu.make_async_copy(dst_hbm.at[s], dst_v, esem).wait()
    pltpu.make_async_copy(widx_hbm.at[c, s], widx_v, esem).wait()

    zeros = jnp.zeros((L,), jnp.float32)

    def zrow(r, _):
        for v in range(H // L):
            buf[r, pl.ds(v * L, L)] = zeros
        return 0

    lax.fori_loop(0, WR, zrow, 0)

    def zfire(k, _):
        pltpu.async_copy(wbuf, acc.at[pl.ds(base + k * WR, WR)], esem)
        return 0

    lax.fori_loop(0, WCH, zfire, 0)

    def zdrain(k, _):
        pltpu.make_async_copy(wbuf, acc.at[pl.ds(base, WR)], esem).wait()
        return 0

    lax.fori_loop(0, WCH, zdrain, 0)
    plsc.subcore_barrier()

    def scale(p):
        def edge_body(e, _):
            w = ewv[pl.ds(e * L, L)]
            for v in range(H // L):
                sl = pl.ds(v * L, L)
                buf[e, sl] = buf[e, sl] * w
            return 0

        lax.fori_loop(0, K, edge_body, 0)

    def chunk_body(j, _):
        pltpu.async_copy(ew_hbm.at[s, j], ewv, esem)
        pltpu.sync_copy(xw2_hbm.at[src_v.at[j]], buf)
        pltpu.make_async_copy(ew_hbm.at[s, 0], ewv, esem).wait()
        scale(0)
        pltpu.sync_copy(buf, acc.at[dst_v.at[j]], add=True)
        return 0

    lax.fori_loop(0, CH, chunk_body, 0)
    plsc.subcore_barrier()

    def wo_body(k, _):
        pltpu.sync_copy(acc.at[pl.ds(base + k * WR, WR)], wbuf)

        def relu_row(r, _):
            for v in range(H // L):
                sl = pl.ds(v * L, L)
                buf[r, sl] = jnp.maximum(buf[r, sl], 0.0)
            return 0

        lax.fori_loop(0, WR, relu_row, 0)
        pltpu.sync_copy(wbuf, out_hbm.at[widx_v.at[k]])
        return 0

    lax.fori_loop(0, WCH, wo_body, 0)


def _sc_scatter(xw2, src2, dst3, ew3, widx):
    mesh = plsc.VectorSubcoreMesh(core_axis_name="c", subcore_axis_name="s",
                                  num_cores=NC, num_subcores=NS)
    return pl.kernel(
        _sc_body,
        out_type=jax.ShapeDtypeStruct((2 * N, H), jnp.float32),
        mesh=mesh,
        scratch_types=[
            pltpu.VMEM((CH, K), jnp.int32),
            pltpu.VMEM((CH, K), jnp.int32),
            pltpu.VMEM((K * L,), jnp.float32),
            pltpu.VMEM((WCH, WR), jnp.int32),
            pltpu.VMEM((K, H), jnp.float32),
            pltpu.VMEM_SHARED((N, H), jnp.float32),
            pltpu.SemaphoreType.DMA,
        ],
    )(xw2, src2, dst3, ew3, widx)


def kernel(x, edge_index, edge_weight, weight):
    xw = _matmul(x, weight)
    xw2 = xw.reshape(2 * N, H)

    pad = NS * EPT - E
    srcp = jnp.pad(edge_index[0], (0, pad))
    dstp = jnp.pad(edge_index[1], (0, pad))
    ewp = jnp.pad(edge_weight, (0, pad))

    core = jnp.arange(NC, dtype=jnp.int32)
    src2 = (2 * srcp[None, :] + core[:, None]).reshape(NC, NS, CH, K)
    dst3 = dstp.reshape(NS, CH, K)
    ew3 = jnp.broadcast_to(ewp[:, None],
                           (NS * EPT, L)).reshape(NS, CH, K * L)

    rows = jnp.arange(N, dtype=jnp.int32).reshape(NS, WCH, WR)
    widx = 2 * rows[None] + core[:, None, None, None]

    out_flat = _sc_scatter(xw2, src2, dst3, ew3, widx)
    return out_flat.reshape(N, D)
